# baseline (speedup 1.0000x reference)
import jax
import jax.numpy as jnp
from jax.experimental import pallas as pl

N = 4096
D = 512
BI = 512
H = N // 2


def _body(nf_ref, e0_ref, e1_ref, a0_ref, a1_ref, o_ref):
    i = pl.program_id(0)
    a0 = a0_ref[...]
    a1 = a1_ref[...]
    o_ref[:D, :] = nf_ref[pl.ds(i * BI, BI), :].T
    nn = jax.lax.dot(
        a0.astype(jnp.bfloat16), nf_ref[:H, :].astype(jnp.bfloat16),
        preferred_element_type=jnp.float32)
    nn += jax.lax.dot(
        a1.astype(jnp.bfloat16), nf_ref[H:, :].astype(jnp.bfloat16),
        preferred_element_type=jnp.float32)
    o_ref[D:2 * D, :] = nn.T
    diag = jnp.sum(a0 * e0_ref[...].T, axis=1, keepdims=True)
    diag += jnp.sum(a1 * e1_ref[...].T, axis=1, keepdims=True)
    o_ref[2 * D:, :] = diag.T


@jax.jit
def kernel(node_features, edge_features, adj):
    out_t = pl.pallas_call(
        _body,
        grid=(N // BI,),
        in_specs=[
            pl.BlockSpec((N, D), lambda i: (0, 0)),
            pl.BlockSpec((H, BI), lambda i: (0, i)),
            pl.BlockSpec((H, BI), lambda i: (1, i)),
            pl.BlockSpec((BI, H), lambda i: (i, 0)),
            pl.BlockSpec((BI, H), lambda i: (i, 1)),
        ],
        out_specs=pl.BlockSpec((2 * D + 1, BI), lambda i: (0, i)),
        out_shape=jax.ShapeDtypeStruct((2 * D + 1, N), jnp.float32),
    )(node_features, edge_features, edge_features, adj, adj)
    return out_t.T


# speedup vs baseline: 1.0157x; 1.0157x over previous
import jax
import jax.numpy as jnp
from jax.experimental import pallas as pl

N = 4096
D = 512
BI = 256
H = N // 2


def _body(nf_ref, e0_ref, e1_ref, a0_ref, a1_ref, o_ref):
    i = pl.program_id(0)
    a0 = a0_ref[...]
    a1 = a1_ref[...]
    o_ref[:D, :] = nf_ref[pl.ds(i * BI, BI), :].T
    nn = jax.lax.dot(
        a0.astype(jnp.bfloat16), nf_ref[:H, :].astype(jnp.bfloat16),
        preferred_element_type=jnp.float32)
    nn += jax.lax.dot(
        a1.astype(jnp.bfloat16), nf_ref[H:, :].astype(jnp.bfloat16),
        preferred_element_type=jnp.float32)
    o_ref[D:2 * D, :] = nn.T
    diag = jnp.sum(a0 * e0_ref[...].T, axis=1, keepdims=True)
    diag += jnp.sum(a1 * e1_ref[...].T, axis=1, keepdims=True)
    o_ref[2 * D:, :] = diag.T


@jax.jit
def kernel(node_features, edge_features, adj):
    out_t = pl.pallas_call(
        _body,
        grid=(N // BI,),
        in_specs=[
            pl.BlockSpec((N, D), lambda i: (0, 0)),
            pl.BlockSpec((H, BI), lambda i: (0, i)),
            pl.BlockSpec((H, BI), lambda i: (1, i)),
            pl.BlockSpec((BI, H), lambda i: (i, 0)),
            pl.BlockSpec((BI, H), lambda i: (i, 1)),
        ],
        out_specs=pl.BlockSpec((2 * D + 1, BI), lambda i: (0, i)),
        out_shape=jax.ShapeDtypeStruct((2 * D + 1, N), jnp.float32),
    )(node_features, edge_features, edge_features, adj, adj)
    return out_t.T


# speedup vs baseline: 1.0274x; 1.0115x over previous
import jax
import jax.numpy as jnp
from jax.experimental import pallas as pl

N = 4096
D = 512
BI = 256
H = N // 2
Q = N // 4


def _body(nf_ref, e0_ref, e1_ref, e2_ref, e3_ref,
          a0_ref, a1_ref, a2_ref, a3_ref, o_ref):
    i = pl.program_id(0)
    o_ref[:D, :] = nf_ref[pl.ds(i * BI, BI), :].T
    nn = None
    diag = None
    for q, (a_ref, e_ref) in enumerate(
            ((a0_ref, e0_ref), (a1_ref, e1_ref),
             (a2_ref, e2_ref), (a3_ref, e3_ref))):
        a = a_ref[...]
        p = jax.lax.dot(
            a.astype(jnp.bfloat16),
            nf_ref[q * Q:(q + 1) * Q, :].astype(jnp.bfloat16),
            preferred_element_type=jnp.float32)
        d = jnp.sum(a * e_ref[...].T, axis=1, keepdims=True)
        nn = p if nn is None else nn + p
        diag = d if diag is None else diag + d
    o_ref[D:2 * D, :] = nn.T
    o_ref[2 * D:, :] = diag.T


@jax.jit
def kernel(node_features, edge_features, adj):
    out_t = pl.pallas_call(
        _body,
        grid=(N // BI,),
        in_specs=[
            pl.BlockSpec((N, D), lambda i: (0, 0)),
            pl.BlockSpec((Q, BI), lambda i: (0, i)),
            pl.BlockSpec((Q, BI), lambda i: (1, i)),
            pl.BlockSpec((Q, BI), lambda i: (2, i)),
            pl.BlockSpec((Q, BI), lambda i: (3, i)),
            pl.BlockSpec((BI, Q), lambda i: (i, 0)),
            pl.BlockSpec((BI, Q), lambda i: (i, 1)),
            pl.BlockSpec((BI, Q), lambda i: (i, 2)),
            pl.BlockSpec((BI, Q), lambda i: (i, 3)),
        ],
        out_specs=pl.BlockSpec((2 * D + 1, BI), lambda i: (0, i)),
        out_shape=jax.ShapeDtypeStruct((2 * D + 1, N), jnp.float32),
    )(node_features, edge_features, edge_features, edge_features, edge_features,
      adj, adj, adj, adj)
    return out_t.T
